# baseline (speedup 1.0000x reference)
import functools
import math

import jax
import jax.numpy as jnp
import numpy as np
from jax.experimental import pallas as pl
from jax.experimental.pallas import tpu as pltpu


def _ru(x, m):
    return (x + m - 1) // m * m


def _same_pads(size, k, stride):
    pad = max((math.ceil(size / stride) - 1) * stride + k - size, 0)
    if size % stride != 0:
        pad += 1
    return pad // 2, pad - pad // 2


SP9 = tuple((0, dh, dw) for dh in (-1, 0, 1) for dw in (-1, 0, 1))
TP3 = tuple((dt, 0, 0) for dt in (-1, 0, 1))
PL27 = tuple((dt, dh, dw) for dt in (-1, 0, 1)
             for dh in (-1, 0, 1) for dw in (-1, 0, 1))


def _stem_kernel(x_ref, w_ref, s_ref, b_ref, o_ref, acc_ref):
    k = pl.program_id(1)

    @pl.when(k == 0)
    def _():
        acc_ref[...] = jnp.zeros_like(acc_ref)

    acc_ref[...] += jnp.dot(x_ref[0, 0], w_ref[0],
                            preferred_element_type=jnp.float32)

    @pl.when(k == pl.num_programs(1) - 1)
    def _():
        o_ref[0] = jnp.maximum(acc_ref[...] * s_ref[...] + b_ref[...], 0.0)


@functools.lru_cache(maxsize=None)
def _build_stem(N, n_taps, R, L, Cl):
    return pl.pallas_call(
        _stem_kernel,
        out_shape=jax.ShapeDtypeStruct((N, R, Cl), jnp.float32),
        grid=(N, n_taps),
        in_specs=[
            pl.BlockSpec((1, 1, R, L), lambda n, k: (k, n, 0, 0)),
            pl.BlockSpec((1, L, Cl), lambda n, k: (k, 0, 0)),
            pl.BlockSpec((1, Cl), lambda n, k: (0, 0)),
            pl.BlockSpec((1, Cl), lambda n, k: (0, 0)),
        ],
        out_specs=pl.BlockSpec((1, R, Cl), lambda n, k: (n, 0, 0)),
        scratch_shapes=[pltpu.VMEM((R, Cl), jnp.float32)],
        compiler_params=pltpu.CompilerParams(
            dimension_semantics=("parallel", "arbitrary")),
    )


def _block_body(kind, n_taps, T, H, W, rows_n, rows_p, widths, Cout_p,
                pool_k, *refs):
    HW = H * W
    o_ref = refs[-1]
    refs = refs[:-1]

    if n_taps:
        x = jnp.max(refs[0][:, 0], axis=0)
    else:
        x = refs[0][0]

    def rowid(width):
        return jax.lax.broadcasted_iota(jnp.int32, (rows_p, width), 0)

    def tapmask(dt, dh, dw, width):
        m = rowid(width)
        ok = m < rows_n
        if dt:
            t = m // HW
            ok &= jnp.logical_and(t + dt >= 0, t + dt < T)
        if dh:
            h = (m // W) % H
            ok &= jnp.logical_and(h + dh >= 0, h + dh < H)
        if dw:
            w = m % W
            ok &= jnp.logical_and(w + dw >= 0, w + dw < W)
        return ok

    def shift(a, r):
        if r == 0:
            return a
        z = jnp.zeros((abs(r), a.shape[1]), a.dtype)
        if r > 0:
            return jnp.concatenate([a[r:], z], axis=0)
        return jnp.concatenate([z, a[:r]], axis=0)

    def tap_val(a, dt, dh, dw):
        r = dt * HW + dh * W + dw
        return jnp.where(tapmask(dt, dh, dw, a.shape[1]), shift(a, r), 0.0)

    def conv_taps(a, taps, w_ref):
        parts = [tap_val(a, *tap) for tap in taps]
        xc = jnp.concatenate(parts, axis=1)
        return jnp.dot(xc, w_ref[...], preferred_element_type=jnp.float32)

    def cbact(y, sb_ref):
        sb = sb_ref[...]
        return jnp.maximum(y * sb[0:1] + sb[1:2], 0.0)

    def gate(a, w_ref):
        valid = rowid(a.shape[1]) < rows_n
        mean = jnp.sum(jnp.where(valid, a, 0.0), axis=0,
                       keepdims=True) * (1.0 / rows_n)
        g = jax.nn.sigmoid(
            jnp.dot(mean, w_ref[...], preferred_element_type=jnp.float32))
        return a * g

    def b1b2(a, r):
        c1w, c1sb, c2w, c2sb, w3, b3, w4 = r
        t = cbact(jnp.dot(a, c1w[...], preferred_element_type=jnp.float32),
                  c1sb)
        t = cbact(conv_taps(t, SP9, c2w), c2sb)
        u = jnp.maximum(conv_taps(t, TP3, w3) + b3[...], 0.0)
        return gate(u, w4)

    def b0b3(a, r, pool):
        c1w, c1sb, gw = r
        if pool:
            acc = None
            for tap in PL27:
                v = tap_val(a, *tap)
                acc = v if acc is None else jnp.maximum(acc, v)
            a = acc
        t = cbact(jnp.dot(a, c1w[...], preferred_element_type=jnp.float32),
                  c1sb)
        return gate(t, gw)

    if kind == "pre":
        y = b1b2(x, refs[1:8])
        cw = y.shape[1]
        if cw < Cout_p:
            y = jnp.concatenate(
                [y, jnp.zeros((rows_p, Cout_p - cw), y.dtype)], axis=1)
    else:
        w0, _, w1, _, w2c, w3c = widths
        y0 = b0b3(x, refs[1:4], False)[:, :w0]
        y1 = b1b2(x, refs[4:11])[:, :w1]
        y2 = b1b2(x, refs[11:18])[:, :w2c]
        y3 = b0b3(x, refs[18:21], True)[:, :w3c]
        parts = [y0, y1, y2, y3]
        tot = w0 + w1 + w2c + w3c
        if tot < Cout_p:
            parts.append(jnp.zeros((rows_p, Cout_p - tot), y0.dtype))
        y = jnp.concatenate(parts, axis=1)
    if pool_k:
        acc = None
        for dt in range(pool_k[0]):
            for dh in range(pool_k[1]):
                for dw in range(pool_k[2]):
                    v = tap_val(y, dt, dh, dw)
                    acc = v if acc is None else jnp.maximum(acc, v)
        y = acc
    o_ref[0] = y


@functools.lru_cache(maxsize=None)
def _build_block(kind, n_taps, N, T, H, W, rows_n, rows_p, Cin_p,
                 widths, Cout_p, wshapes, pool_k):
    body = functools.partial(_block_body, kind, n_taps, T, H, W,
                             rows_n, rows_p, widths, Cout_p, pool_k)
    if n_taps:
        x_spec = pl.BlockSpec((n_taps, 1, rows_p, Cin_p),
                              lambda n: (0, n, 0, 0))
    else:
        x_spec = pl.BlockSpec((1, rows_p, Cin_p), lambda n: (n, 0, 0))
    in_specs = [x_spec] + [pl.BlockSpec(s, lambda n: (0, 0)) for s in wshapes]
    return pl.pallas_call(
        body,
        out_shape=jax.ShapeDtypeStruct((N, rows_p, Cout_p), jnp.float32),
        grid=(N,),
        in_specs=in_specs,
        out_specs=pl.BlockSpec((1, rows_p, Cout_p), lambda n: (n, 0, 0)),
        compiler_params=pltpu.CompilerParams(
            dimension_semantics=("parallel",)),
    )


def _p2(a, r, c):
    return jnp.pad(a, ((0, r - a.shape[0]), (0, c - a.shape[1])))


def _pv(v, c):
    return jnp.pad(v.astype(jnp.float32), (0, c - v.shape[0])).reshape(1, c)


def _psb(s, b, cp):
    sb = jnp.stack([s.astype(jnp.float32), b.astype(jnp.float32)])
    return jnp.pad(sb, ((0, 0), (0, cp - sb.shape[1])))


def _stack5(w, cin_p, cout_p):
    kT, kH, kW, cin, cout = w.shape
    wp = jnp.pad(w, ((0, 0), (0, 0), (0, 0),
                     (0, cin_p - cin), (0, cout_p - cout)))
    return wp.reshape(kT * kH * kW * cin_p, cout_p)


def _prep_b1b2(cin_p, c1w, c1s, c1b, c2w, c2s, c2b, w3, b3, w4):
    c1 = c1w.shape[-1]
    c2 = c2w.shape[-1]
    c1p = _ru(c1, 128)
    c2p = _ru(c2, 128)
    arrs = [
        _stack5(c1w, cin_p, c1p), _psb(c1s, c1b, c1p),
        _stack5(c2w, c1p, c2p), _psb(c2s, c2b, c2p),
        _stack5(w3, c2p, c2p), _pv(b3, c2p),
        _stack5(w4, c2p, c2p),
    ]
    return arrs, c2


def _prep_b0b3(cin_p, c1w, c1s, c1b, gw):
    c = c1w.shape[-1]
    cp = _ru(c, 128)
    arrs = [_stack5(c1w, cin_p, cp), _psb(c1s, c1b, cp),
            _stack5(gw.reshape(1, 1, 1, c, c), cp, cp)]
    return arrs, c


def _pool_taps(y, ksize, stride):
    N, T, H, W, C = y.shape
    pads = [(0, 0)]
    for i, (dim, k, s) in enumerate(zip((T, H, W), ksize, stride)):
        pads.append(_same_pads(dim, k, s))
    pads.append((0, 0))
    yp = jnp.pad(y, pads)
    Tp, Hp, Wp = yp.shape[1:4]
    To = (Tp - ksize[0]) // stride[0] + 1
    Ho = (Hp - ksize[1]) // stride[1] + 1
    Wo = (Wp - ksize[2]) // stride[2] + 1
    taps = []
    for dt in range(ksize[0]):
        for dh in range(ksize[1]):
            for dw in range(ksize[2]):
                taps.append(yp[:, dt:dt + stride[0] * To:stride[0],
                               dh:dh + stride[1] * Ho:stride[1],
                               dw:dw + stride[2] * Wo:stride[2], :])
    rows = To * Ho * Wo
    rows_p = _ru(rows, 8)
    st = jnp.stack(taps, axis=0).reshape(len(taps), N, rows, C)
    st = jnp.pad(st, ((0, 0), (0, 0), (0, rows_p - rows), (0, 0)))
    return st, (To, Ho, Wo, rows, rows_p)


def _run_block(kind, x, n_taps, dims, params, pool_k=()):
    T, H, W, rows_n, rows_p = dims
    if n_taps:
        N = x.shape[1]
        Cin_p = x.shape[3]
    else:
        N = x.shape[0]
        Cin_p = x.shape[2]
    if kind == "pre":
        arrs, c2 = _prep_b1b2(Cin_p, *params)
        widths = (c2,)
        Cout_p = _ru(c2, 128)
    else:
        b0a, b0c = _prep_b0b3(Cin_p, *params[0])
        b1a, b1c = _prep_b1b2(Cin_p, *params[1])
        b2a, b2c = _prep_b1b2(Cin_p, *params[2])
        b3a, b3c = _prep_b0b3(Cin_p, *params[3])
        arrs = b0a + b1a + b2a + b3a
        widths = (b0c, 0, b1c, 0, b2c, b3c)
        Cout_p = _ru(b0c + b1c + b2c + b3c, 128)
    wshapes = tuple(a.shape for a in arrs)
    fn = _build_block(kind, n_taps, N, T, H, W, rows_n, rows_p, Cin_p,
                      widths, Cout_p, wshapes, pool_k)
    out = fn(x, *arrs)
    return out, b0c + b1c + b2c + b3c if kind != "pre" else out.shape[-1]


@functools.lru_cache(maxsize=None)
def _stem_onehot(Wo, kW, Cin, Lp):
    oh = np.zeros((Lp, Wo * kW * Cin), np.float32)
    for wo in range(Wo):
        for dw in range(kW):
            for c in range(Cin):
                l = (2 * wo + dw) * Cin + c
                oh[l, (wo * kW + dw) * Cin + c] = 1.0
    return jnp.asarray(oh)


def _subsample(y, dims, stride):
    T, H, W, rows, rows_p = dims
    N, _, C = y.shape
    y5 = y[:, :rows].reshape(N, T, H, W, C)
    y5 = y5[:, ::stride[0], ::stride[1], ::stride[2]]
    To, Ho, Wo = y5.shape[1:4]
    r2 = To * Ho * Wo
    r2p = _ru(r2, 8)
    y2 = y5.reshape(N, r2, C)
    if r2p != r2:
        y2 = jnp.pad(y2, ((0, 0), (0, r2p - r2), (0, 0)))
    return y2, (To, Ho, Wo, r2, r2p)


def _conv_stem(x, w, s, b):
    N, T, H, W, Cin = x.shape
    kT, kH, kW, _, Cout = w.shape
    pt = _same_pads(T, kT, 2)
    ph = _same_pads(H, kH, 2)
    pw = _same_pads(W, kW, 2)
    xw = x.reshape(N, T, H, W * Cin)
    xw = jnp.pad(xw, ((0, 0), pt, ph, (pw[0] * Cin, pw[1] * Cin)))
    Tp, Hp = xw.shape[1:3]
    L = xw.shape[3]
    To = (Tp - kT) // 2 + 1
    Ho = (Hp - kH) // 2 + 1
    Wp = L // Cin
    Wo = (Wp - kW) // 2 + 1
    Lp = _ru(L, 128)
    xw = jnp.pad(xw, ((0, 0), (0, 0), (0, 0), (0, Lp - L)))
    taps = []
    for dt in range(kT):
        for dh in range(kH):
            taps.append(xw[:, dt:dt + 2 * To:2, dh:dh + 2 * Ho:2, :])
    n_taps = len(taps)
    R = To * Ho
    xp = jnp.stack(taps, axis=0).reshape(n_taps, N, R, Lp)
    xp = xp.astype(jnp.bfloat16)
    oh = _stem_onehot(Wo, kW, Cin, Lp)
    wf = w.reshape(kT * kH, kW * Cin, Cout)
    wbig = jnp.einsum("lwk,tkc->tlwc", oh.reshape(Lp, Wo, kW * Cin), wf)
    wbig = wbig.reshape(n_taps, Lp, Wo * Cout).astype(jnp.bfloat16)
    sv = jnp.tile(s.astype(jnp.float32), Wo).reshape(1, Wo * Cout)
    bv = jnp.tile(b.astype(jnp.float32), Wo).reshape(1, Wo * Cout)
    y = _build_stem(N, n_taps, R, Lp, Wo * Cout)(xp, wbig, sv, bv)
    return y.reshape(N, To, Ho, Wo, Cout)


def kernel(stem_w, stem_s, stem_b, pre_conv1_w, pre_conv1_s, pre_conv1_b, pre_conv2_w, pre_conv2_s, pre_conv2_b, pre_w3, pre_b3, pre_w4, m0_b0_conv1_w, m0_b0_conv1_s, m0_b0_conv1_b, m0_b0_w2, m0_b1_conv1_w, m0_b1_conv1_s, m0_b1_conv1_b, m0_b1_conv2_w, m0_b1_conv2_s, m0_b1_conv2_b, m0_b1_w3, m0_b1_b3, m0_b1_w4, m0_b2_conv1_w, m0_b2_conv1_s, m0_b2_conv1_b, m0_b2_conv2_w, m0_b2_conv2_s, m0_b2_conv2_b, m0_b2_w3, m0_b2_b3, m0_b2_w4, m0_b3_conv1_w, m0_b3_conv1_s, m0_b3_conv1_b, m0_b3_w2, m1_b0_conv1_w, m1_b0_conv1_s, m1_b0_conv1_b, m1_b0_w2, m1_b1_conv1_w, m1_b1_conv1_s, m1_b1_conv1_b, m1_b1_conv2_w, m1_b1_conv2_s, m1_b1_conv2_b, m1_b1_w3, m1_b1_b3, m1_b1_w4, m1_b2_conv1_w, m1_b2_conv1_s, m1_b2_conv1_b, m1_b2_conv2_w, m1_b2_conv2_s, m1_b2_conv2_b, m1_b2_w3, m1_b2_b3, m1_b2_w4, m1_b3_conv1_w, m1_b3_conv1_s, m1_b3_conv1_b, m1_b3_w2, m2_b0_conv1_w, m2_b0_conv1_s, m2_b0_conv1_b, m2_b0_w2, m2_b1_conv1_w, m2_b1_conv1_s, m2_b1_conv1_b, m2_b1_conv2_w, m2_b1_conv2_s, m2_b1_conv2_b, m2_b1_w3, m2_b1_b3, m2_b1_w4, m2_b2_conv1_w, m2_b2_conv1_s, m2_b2_conv1_b, m2_b2_conv2_w, m2_b2_conv2_s, m2_b2_conv2_b, m2_b2_w3, m2_b2_b3, m2_b2_w4, m2_b3_conv1_w, m2_b3_conv1_s, m2_b3_conv1_b, m2_b3_w2, m3_b0_conv1_w, m3_b0_conv1_s, m3_b0_conv1_b, m3_b0_w2, m3_b1_conv1_w, m3_b1_conv1_s, m3_b1_conv1_b, m3_b1_conv2_w, m3_b1_conv2_s, m3_b1_conv2_b, m3_b1_w3, m3_b1_b3, m3_b1_w4, m3_b2_conv1_w, m3_b2_conv1_s, m3_b2_conv1_b, m3_b2_conv2_w, m3_b2_conv2_s, m3_b2_conv2_b, m3_b2_w3, m3_b2_b3, m3_b2_w4, m3_b3_conv1_w, m3_b3_conv1_s, m3_b3_conv1_b, m3_b3_w2, m4_b0_conv1_w, m4_b0_conv1_s, m4_b0_conv1_b, m4_b0_w2, m4_b1_conv1_w, m4_b1_conv1_s, m4_b1_conv1_b, m4_b1_conv2_w, m4_b1_conv2_s, m4_b1_conv2_b, m4_b1_w3, m4_b1_b3, m4_b1_w4, m4_b2_conv1_w, m4_b2_conv1_s, m4_b2_conv1_b, m4_b2_conv2_w, m4_b2_conv2_s, m4_b2_conv2_b, m4_b2_w3, m4_b2_b3, m4_b2_w4, m4_b3_conv1_w, m4_b3_conv1_s, m4_b3_conv1_b, m4_b3_w2, m5_b0_conv1_w, m5_b0_conv1_s, m5_b0_conv1_b, m5_b0_w2, m5_b1_conv1_w, m5_b1_conv1_s, m5_b1_conv1_b, m5_b1_conv2_w, m5_b1_conv2_s, m5_b1_conv2_b, m5_b1_w3, m5_b1_b3, m5_b1_w4, m5_b2_conv1_w, m5_b2_conv1_s, m5_b2_conv1_b, m5_b2_conv2_w, m5_b2_conv2_s, m5_b2_conv2_b, m5_b2_w3, m5_b2_b3, m5_b2_w4, m5_b3_conv1_w, m5_b3_conv1_s, m5_b3_conv1_b, m5_b3_w2, m6_b0_conv1_w, m6_b0_conv1_s, m6_b0_conv1_b, m6_b0_w2, m6_b1_conv1_w, m6_b1_conv1_s, m6_b1_conv1_b, m6_b1_conv2_w, m6_b1_conv2_s, m6_b1_conv2_b, m6_b1_w3, m6_b1_b3, m6_b1_w4, m6_b2_conv1_w, m6_b2_conv1_s, m6_b2_conv1_b, m6_b2_conv2_w, m6_b2_conv2_s, m6_b2_conv2_b, m6_b2_w3, m6_b2_b3, m6_b2_w4, m6_b3_conv1_w, m6_b3_conv1_s, m6_b3_conv1_b, m6_b3_w2, m7_b0_conv1_w, m7_b0_conv1_s, m7_b0_conv1_b, m7_b0_w2, m7_b1_conv1_w, m7_b1_conv1_s, m7_b1_conv1_b, m7_b1_conv2_w, m7_b1_conv2_s, m7_b1_conv2_b, m7_b1_w3, m7_b1_b3, m7_b1_w4, m7_b2_conv1_w, m7_b2_conv1_s, m7_b2_conv1_b, m7_b2_conv2_w, m7_b2_conv2_s, m7_b2_conv2_b, m7_b2_w3, m7_b2_b3, m7_b2_w4, m7_b3_conv1_w, m7_b3_conv1_s, m7_b3_conv1_b, m7_b3_w2, m8_b0_conv1_w, m8_b0_conv1_s, m8_b0_conv1_b, m8_b0_w2, m8_b1_conv1_w, m8_b1_conv1_s, m8_b1_conv1_b, m8_b1_conv2_w, m8_b1_conv2_s, m8_b1_conv2_b, m8_b1_w3, m8_b1_b3, m8_b1_w4, m8_b2_conv1_w, m8_b2_conv1_s, m8_b2_conv1_b, m8_b2_conv2_w, m8_b2_conv2_s, m8_b2_conv2_b, m8_b2_w3, m8_b2_b3, m8_b2_w4, m8_b3_conv1_w, m8_b3_conv1_s, m8_b3_conv1_b, m8_b3_w2, x):
    ml = locals()
    mixed = []
    for i in range(9):
        p = f"m{i}_"
        mixed.append((
            (ml[p + "b0_conv1_w"], ml[p + "b0_conv1_s"],
             ml[p + "b0_conv1_b"], ml[p + "b0_w2"]),
            (ml[p + "b1_conv1_w"], ml[p + "b1_conv1_s"], ml[p + "b1_conv1_b"],
             ml[p + "b1_conv2_w"], ml[p + "b1_conv2_s"], ml[p + "b1_conv2_b"],
             ml[p + "b1_w3"], ml[p + "b1_b3"], ml[p + "b1_w4"]),
            (ml[p + "b2_conv1_w"], ml[p + "b2_conv1_s"], ml[p + "b2_conv1_b"],
             ml[p + "b2_conv2_w"], ml[p + "b2_conv2_s"], ml[p + "b2_conv2_b"],
             ml[p + "b2_w3"], ml[p + "b2_b3"], ml[p + "b2_w4"]),
            (ml[p + "b3_conv1_w"], ml[p + "b3_conv1_s"],
             ml[p + "b3_conv1_b"], ml[p + "b3_w2"]),
        ))

    N = x.shape[0]
    y = _conv_stem(x, stem_w, stem_s, stem_b)

    taps, dims = _pool_taps(y, (1, 3, 3), (1, 2, 2))
    T, H, W, rows, rows_p = dims
    pre_params = (pre_conv1_w, pre_conv1_s, pre_conv1_b, pre_conv2_w,
                  pre_conv2_s, pre_conv2_b, pre_w3, pre_b3, pre_w4)
    y, _ = _run_block("pre", taps, taps.shape[0], dims, pre_params,
                      pool_k=(1, 3, 3))
    y, dims = _subsample(y, dims, (1, 2, 2))

    y, _ = _run_block("mixed", y, 0, dims, mixed[0])
    y, _ = _run_block("mixed", y, 0, dims, mixed[1], pool_k=(3, 3, 3))
    y, dims = _subsample(y, dims, (2, 2, 2))

    y, _ = _run_block("mixed", y, 0, dims, mixed[2])
    for i in range(3, 6):
        y, _ = _run_block("mixed", y, 0, dims, mixed[i])
    y, _ = _run_block("mixed", y, 0, dims, mixed[6], pool_k=(2, 2, 2))
    y, dims = _subsample(y, dims, (2, 2, 2))

    y, _ = _run_block("mixed", y, 0, dims, mixed[7])
    y, c = _run_block("mixed", y, 0, dims, mixed[8])
    T, H, W, rows, rows_p = dims
    return y[:, :rows, :c].reshape(N, T, H, W, c)


# speedup vs baseline: 1.0141x; 1.0141x over previous
import functools
import math

import jax
import jax.numpy as jnp
import numpy as np
from jax.experimental import pallas as pl
from jax.experimental.pallas import tpu as pltpu


def _ru(x, m):
    return (x + m - 1) // m * m


def _same_pads(size, k, stride):
    pad = max((math.ceil(size / stride) - 1) * stride + k - size, 0)
    if size % stride != 0:
        pad += 1
    return pad // 2, pad - pad // 2


SP9 = tuple((0, dh, dw) for dh in (-1, 0, 1) for dw in (-1, 0, 1))
TP3 = tuple((dt, 0, 0) for dt in (-1, 0, 1))
PL27 = tuple((dt, dh, dw) for dt in (-1, 0, 1)
             for dh in (-1, 0, 1) for dw in (-1, 0, 1))


def _stem_kernel(x_ref, w_ref, s_ref, b_ref, o_ref, acc_ref):
    k = pl.program_id(1)

    @pl.when(k == 0)
    def _():
        acc_ref[...] = jnp.zeros_like(acc_ref)

    acc_ref[...] += jnp.dot(x_ref[0, 0], w_ref[0],
                            preferred_element_type=jnp.float32)

    @pl.when(k == pl.num_programs(1) - 1)
    def _():
        o_ref[0] = jnp.maximum(acc_ref[...] * s_ref[...] + b_ref[...], 0.0)


@functools.lru_cache(maxsize=None)
def _build_stem(N, n_taps, R, L, Cl):
    return pl.pallas_call(
        _stem_kernel,
        out_shape=jax.ShapeDtypeStruct((N, R, Cl), jnp.float32),
        grid=(N, n_taps),
        in_specs=[
            pl.BlockSpec((1, 1, R, L), lambda n, k: (k, n, 0, 0)),
            pl.BlockSpec((1, L, Cl), lambda n, k: (k, 0, 0)),
            pl.BlockSpec((1, Cl), lambda n, k: (0, 0)),
            pl.BlockSpec((1, Cl), lambda n, k: (0, 0)),
        ],
        out_specs=pl.BlockSpec((1, R, Cl), lambda n, k: (n, 0, 0)),
        scratch_shapes=[pltpu.VMEM((R, Cl), jnp.float32)],
        compiler_params=pltpu.CompilerParams(
            dimension_semantics=("parallel", "arbitrary")),
    )


def _block_body(blocks, n_taps, T, H, W, rows_n, rows_p, *refs):
    HW = H * W
    o_ref = refs[-1]
    refs = refs[:-1]

    if n_taps:
        x = jnp.max(refs[0][:, 0], axis=0)
    else:
        x = refs[0][0]

    def rowid(width):
        return jax.lax.broadcasted_iota(jnp.int32, (rows_p, width), 0)

    def tapmask(dt, dh, dw, width):
        m = rowid(width)
        ok = m < rows_n
        if dt:
            t = m // HW
            ok &= jnp.logical_and(t + dt >= 0, t + dt < T)
        if dh:
            h = (m // W) % H
            ok &= jnp.logical_and(h + dh >= 0, h + dh < H)
        if dw:
            w = m % W
            ok &= jnp.logical_and(w + dw >= 0, w + dw < W)
        return ok

    def shift(a, r):
        if r == 0:
            return a
        z = jnp.zeros((abs(r), a.shape[1]), a.dtype)
        if r > 0:
            return jnp.concatenate([a[r:], z], axis=0)
        return jnp.concatenate([z, a[:r]], axis=0)

    def tap_val(a, dt, dh, dw):
        r = dt * HW + dh * W + dw
        return jnp.where(tapmask(dt, dh, dw, a.shape[1]), shift(a, r), 0.0)

    def conv_taps(a, taps, w_ref):
        parts = [tap_val(a, *tap) for tap in taps]
        xc = jnp.concatenate(parts, axis=1)
        return jnp.dot(xc, w_ref[...], preferred_element_type=jnp.float32)

    def cbact(y, sb_ref):
        sb = sb_ref[...]
        return jnp.maximum(y * sb[0:1] + sb[1:2], 0.0)

    def gate(a, w_ref):
        valid = rowid(a.shape[1]) < rows_n
        mean = jnp.sum(jnp.where(valid, a, 0.0), axis=0,
                       keepdims=True) * (1.0 / rows_n)
        g = jax.nn.sigmoid(
            jnp.dot(mean, w_ref[...], preferred_element_type=jnp.float32))
        return a * g

    def b1b2(a, r):
        c1w, c1sb, c2w, c2sb, w3, b3, w4 = r
        t = cbact(jnp.dot(a, c1w[...], preferred_element_type=jnp.float32),
                  c1sb)
        t = cbact(conv_taps(t, SP9, c2w), c2sb)
        u = jnp.maximum(conv_taps(t, TP3, w3) + b3[...], 0.0)
        return gate(u, w4)

    def b0b3(a, r, pool):
        c1w, c1sb, gw = r
        if pool:
            acc = None
            for tap in PL27:
                v = tap_val(a, *tap)
                acc = v if acc is None else jnp.maximum(acc, v)
            a = acc
        t = cbact(jnp.dot(a, c1w[...], preferred_element_type=jnp.float32),
                  c1sb)
        return gate(t, gw)

    i = 1
    for kind, widths, Cout_p, pool_k in blocks:
        if kind == "pre":
            y = b1b2(x, refs[i:i + 7])
            i += 7
            cw = y.shape[1]
            if cw < Cout_p:
                y = jnp.concatenate(
                    [y, jnp.zeros((rows_p, Cout_p - cw), y.dtype)], axis=1)
        else:
            w0, _, w1, _, w2c, w3c = widths
            y0 = b0b3(x, refs[i:i + 3], False)[:, :w0]
            y1 = b1b2(x, refs[i + 3:i + 10])[:, :w1]
            y2 = b1b2(x, refs[i + 10:i + 17])[:, :w2c]
            y3 = b0b3(x, refs[i + 17:i + 20], True)[:, :w3c]
            i += 20
            parts = [y0, y1, y2, y3]
            tot = w0 + w1 + w2c + w3c
            if tot < Cout_p:
                parts.append(jnp.zeros((rows_p, Cout_p - tot), y0.dtype))
            y = jnp.concatenate(parts, axis=1)
        if pool_k:
            acc = None
            for dt in range(pool_k[0]):
                for dh in range(pool_k[1]):
                    for dw in range(pool_k[2]):
                        v = tap_val(y, dt, dh, dw)
                        acc = v if acc is None else jnp.maximum(acc, v)
            y = acc
        x = y
    o_ref[0] = x


@functools.lru_cache(maxsize=None)
def _build_block(blocks, n_taps, N, T, H, W, rows_n, rows_p, Cin_p, wshapes):
    body = functools.partial(_block_body, blocks, n_taps, T, H, W,
                             rows_n, rows_p)
    if n_taps:
        x_spec = pl.BlockSpec((n_taps, 1, rows_p, Cin_p),
                              lambda n: (0, n, 0, 0))
    else:
        x_spec = pl.BlockSpec((1, rows_p, Cin_p), lambda n: (n, 0, 0))
    in_specs = [x_spec] + [pl.BlockSpec(s, lambda n: (0, 0)) for s in wshapes]
    Cout_p = blocks[-1][2]
    return pl.pallas_call(
        body,
        out_shape=jax.ShapeDtypeStruct((N, rows_p, Cout_p), jnp.float32),
        grid=(N,),
        in_specs=in_specs,
        out_specs=pl.BlockSpec((1, rows_p, Cout_p), lambda n: (n, 0, 0)),
        compiler_params=pltpu.CompilerParams(
            dimension_semantics=("parallel",)),
    )


def _p2(a, r, c):
    return jnp.pad(a, ((0, r - a.shape[0]), (0, c - a.shape[1])))


def _pv(v, c):
    return jnp.pad(v.astype(jnp.float32), (0, c - v.shape[0])).reshape(1, c)


def _psb(s, b, cp):
    sb = jnp.stack([s.astype(jnp.float32), b.astype(jnp.float32)])
    return jnp.pad(sb, ((0, 0), (0, cp - sb.shape[1])))


def _stack5(w, cin_p, cout_p):
    kT, kH, kW, cin, cout = w.shape
    wp = jnp.pad(w, ((0, 0), (0, 0), (0, 0),
                     (0, cin_p - cin), (0, cout_p - cout)))
    return wp.reshape(kT * kH * kW * cin_p, cout_p)


def _prep_b1b2(cin_p, c1w, c1s, c1b, c2w, c2s, c2b, w3, b3, w4):
    c1 = c1w.shape[-1]
    c2 = c2w.shape[-1]
    c1p = _ru(c1, 128)
    c2p = _ru(c2, 128)
    arrs = [
        _stack5(c1w, cin_p, c1p), _psb(c1s, c1b, c1p),
        _stack5(c2w, c1p, c2p), _psb(c2s, c2b, c2p),
        _stack5(w3, c2p, c2p), _pv(b3, c2p),
        _stack5(w4, c2p, c2p),
    ]
    return arrs, c2


def _prep_b0b3(cin_p, c1w, c1s, c1b, gw):
    c = c1w.shape[-1]
    cp = _ru(c, 128)
    arrs = [_stack5(c1w, cin_p, cp), _psb(c1s, c1b, cp),
            _stack5(gw.reshape(1, 1, 1, c, c), cp, cp)]
    return arrs, c


def _pool_taps(y, ksize, stride):
    N, T, H, W, C = y.shape
    pads = [(0, 0)]
    for i, (dim, k, s) in enumerate(zip((T, H, W), ksize, stride)):
        pads.append(_same_pads(dim, k, s))
    pads.append((0, 0))
    yp = jnp.pad(y, pads)
    Tp, Hp, Wp = yp.shape[1:4]
    To = (Tp - ksize[0]) // stride[0] + 1
    Ho = (Hp - ksize[1]) // stride[1] + 1
    Wo = (Wp - ksize[2]) // stride[2] + 1
    taps = []
    for dt in range(ksize[0]):
        for dh in range(ksize[1]):
            for dw in range(ksize[2]):
                taps.append(yp[:, dt:dt + stride[0] * To:stride[0],
                               dh:dh + stride[1] * Ho:stride[1],
                               dw:dw + stride[2] * Wo:stride[2], :])
    rows = To * Ho * Wo
    rows_p = _ru(rows, 8)
    st = jnp.stack(taps, axis=0).reshape(len(taps), N, rows, C)
    st = jnp.pad(st, ((0, 0), (0, 0), (0, rows_p - rows), (0, 0)))
    return st, (To, Ho, Wo, rows, rows_p)


def _run_blocks(x, n_taps, dims, specs):
    T, H, W, rows_n, rows_p = dims
    if n_taps:
        N = x.shape[1]
        Cin_p = x.shape[3]
    else:
        N = x.shape[0]
        Cin_p = x.shape[2]
    arrs = []
    blocks = []
    cw = Cin_p
    c_true = None
    for kind, params, pool_k in specs:
        if kind == "pre":
            a, c2 = _prep_b1b2(cw, *params)
            widths = (c2,)
            c_true = c2
        else:
            b0a, b0c = _prep_b0b3(cw, *params[0])
            b1a, b1c = _prep_b1b2(cw, *params[1])
            b2a, b2c = _prep_b1b2(cw, *params[2])
            b3a, b3c = _prep_b0b3(cw, *params[3])
            a = b0a + b1a + b2a + b3a
            widths = (b0c, 0, b1c, 0, b2c, b3c)
            c_true = b0c + b1c + b2c + b3c
        cw = _ru(c_true, 128)
        arrs.extend(a)
        blocks.append((kind, widths, cw, pool_k if pool_k else ()))
    wshapes = tuple(a.shape for a in arrs)
    fn = _build_block(tuple(blocks), n_taps, N, T, H, W, rows_n, rows_p,
                      Cin_p, wshapes)
    return fn(x, *arrs), c_true


@functools.lru_cache(maxsize=None)
def _stem_onehot(Wo, kW, Cin, Lp):
    oh = np.zeros((Lp, Wo * kW * Cin), np.float32)
    for wo in range(Wo):
        for dw in range(kW):
            for c in range(Cin):
                l = (2 * wo + dw) * Cin + c
                oh[l, (wo * kW + dw) * Cin + c] = 1.0
    return jnp.asarray(oh)


def _subsample(y, dims, stride):
    T, H, W, rows, rows_p = dims
    N, _, C = y.shape
    y5 = y[:, :rows].reshape(N, T, H, W, C)
    y5 = y5[:, ::stride[0], ::stride[1], ::stride[2]]
    To, Ho, Wo = y5.shape[1:4]
    r2 = To * Ho * Wo
    r2p = _ru(r2, 8)
    y2 = y5.reshape(N, r2, C)
    if r2p != r2:
        y2 = jnp.pad(y2, ((0, 0), (0, r2p - r2), (0, 0)))
    return y2, (To, Ho, Wo, r2, r2p)


def _conv_stem(x, w, s, b):
    N, T, H, W, Cin = x.shape
    kT, kH, kW, _, Cout = w.shape
    pt = _same_pads(T, kT, 2)
    ph = _same_pads(H, kH, 2)
    pw = _same_pads(W, kW, 2)
    xw = x.reshape(N, T, H, W * Cin)
    xw = jnp.pad(xw, ((0, 0), pt, ph, (pw[0] * Cin, pw[1] * Cin)))
    Tp, Hp = xw.shape[1:3]
    L = xw.shape[3]
    To = (Tp - kT) // 2 + 1
    Ho = (Hp - kH) // 2 + 1
    Wp = L // Cin
    Wo = (Wp - kW) // 2 + 1
    Lp = _ru(L, 128)
    xw = jnp.pad(xw, ((0, 0), (0, 0), (0, 0), (0, Lp - L)))
    taps = []
    for dt in range(kT):
        for dh in range(kH):
            taps.append(xw[:, dt:dt + 2 * To:2, dh:dh + 2 * Ho:2, :])
    n_taps = len(taps)
    R = To * Ho
    xp = jnp.stack(taps, axis=0).reshape(n_taps, N, R, Lp)
    xp = xp.astype(jnp.bfloat16)
    oh = _stem_onehot(Wo, kW, Cin, Lp)
    wf = w.reshape(kT * kH, kW * Cin, Cout)
    wbig = jnp.einsum("lwk,tkc->tlwc", oh.reshape(Lp, Wo, kW * Cin), wf)
    wbig = wbig.reshape(n_taps, Lp, Wo * Cout).astype(jnp.bfloat16)
    sv = jnp.tile(s.astype(jnp.float32), Wo).reshape(1, Wo * Cout)
    bv = jnp.tile(b.astype(jnp.float32), Wo).reshape(1, Wo * Cout)
    y = _build_stem(N, n_taps, R, Lp, Wo * Cout)(xp, wbig, sv, bv)
    return y.reshape(N, To, Ho, Wo, Cout)


def kernel(stem_w, stem_s, stem_b, pre_conv1_w, pre_conv1_s, pre_conv1_b, pre_conv2_w, pre_conv2_s, pre_conv2_b, pre_w3, pre_b3, pre_w4, m0_b0_conv1_w, m0_b0_conv1_s, m0_b0_conv1_b, m0_b0_w2, m0_b1_conv1_w, m0_b1_conv1_s, m0_b1_conv1_b, m0_b1_conv2_w, m0_b1_conv2_s, m0_b1_conv2_b, m0_b1_w3, m0_b1_b3, m0_b1_w4, m0_b2_conv1_w, m0_b2_conv1_s, m0_b2_conv1_b, m0_b2_conv2_w, m0_b2_conv2_s, m0_b2_conv2_b, m0_b2_w3, m0_b2_b3, m0_b2_w4, m0_b3_conv1_w, m0_b3_conv1_s, m0_b3_conv1_b, m0_b3_w2, m1_b0_conv1_w, m1_b0_conv1_s, m1_b0_conv1_b, m1_b0_w2, m1_b1_conv1_w, m1_b1_conv1_s, m1_b1_conv1_b, m1_b1_conv2_w, m1_b1_conv2_s, m1_b1_conv2_b, m1_b1_w3, m1_b1_b3, m1_b1_w4, m1_b2_conv1_w, m1_b2_conv1_s, m1_b2_conv1_b, m1_b2_conv2_w, m1_b2_conv2_s, m1_b2_conv2_b, m1_b2_w3, m1_b2_b3, m1_b2_w4, m1_b3_conv1_w, m1_b3_conv1_s, m1_b3_conv1_b, m1_b3_w2, m2_b0_conv1_w, m2_b0_conv1_s, m2_b0_conv1_b, m2_b0_w2, m2_b1_conv1_w, m2_b1_conv1_s, m2_b1_conv1_b, m2_b1_conv2_w, m2_b1_conv2_s, m2_b1_conv2_b, m2_b1_w3, m2_b1_b3, m2_b1_w4, m2_b2_conv1_w, m2_b2_conv1_s, m2_b2_conv1_b, m2_b2_conv2_w, m2_b2_conv2_s, m2_b2_conv2_b, m2_b2_w3, m2_b2_b3, m2_b2_w4, m2_b3_conv1_w, m2_b3_conv1_s, m2_b3_conv1_b, m2_b3_w2, m3_b0_conv1_w, m3_b0_conv1_s, m3_b0_conv1_b, m3_b0_w2, m3_b1_conv1_w, m3_b1_conv1_s, m3_b1_conv1_b, m3_b1_conv2_w, m3_b1_conv2_s, m3_b1_conv2_b, m3_b1_w3, m3_b1_b3, m3_b1_w4, m3_b2_conv1_w, m3_b2_conv1_s, m3_b2_conv1_b, m3_b2_conv2_w, m3_b2_conv2_s, m3_b2_conv2_b, m3_b2_w3, m3_b2_b3, m3_b2_w4, m3_b3_conv1_w, m3_b3_conv1_s, m3_b3_conv1_b, m3_b3_w2, m4_b0_conv1_w, m4_b0_conv1_s, m4_b0_conv1_b, m4_b0_w2, m4_b1_conv1_w, m4_b1_conv1_s, m4_b1_conv1_b, m4_b1_conv2_w, m4_b1_conv2_s, m4_b1_conv2_b, m4_b1_w3, m4_b1_b3, m4_b1_w4, m4_b2_conv1_w, m4_b2_conv1_s, m4_b2_conv1_b, m4_b2_conv2_w, m4_b2_conv2_s, m4_b2_conv2_b, m4_b2_w3, m4_b2_b3, m4_b2_w4, m4_b3_conv1_w, m4_b3_conv1_s, m4_b3_conv1_b, m4_b3_w2, m5_b0_conv1_w, m5_b0_conv1_s, m5_b0_conv1_b, m5_b0_w2, m5_b1_conv1_w, m5_b1_conv1_s, m5_b1_conv1_b, m5_b1_conv2_w, m5_b1_conv2_s, m5_b1_conv2_b, m5_b1_w3, m5_b1_b3, m5_b1_w4, m5_b2_conv1_w, m5_b2_conv1_s, m5_b2_conv1_b, m5_b2_conv2_w, m5_b2_conv2_s, m5_b2_conv2_b, m5_b2_w3, m5_b2_b3, m5_b2_w4, m5_b3_conv1_w, m5_b3_conv1_s, m5_b3_conv1_b, m5_b3_w2, m6_b0_conv1_w, m6_b0_conv1_s, m6_b0_conv1_b, m6_b0_w2, m6_b1_conv1_w, m6_b1_conv1_s, m6_b1_conv1_b, m6_b1_conv2_w, m6_b1_conv2_s, m6_b1_conv2_b, m6_b1_w3, m6_b1_b3, m6_b1_w4, m6_b2_conv1_w, m6_b2_conv1_s, m6_b2_conv1_b, m6_b2_conv2_w, m6_b2_conv2_s, m6_b2_conv2_b, m6_b2_w3, m6_b2_b3, m6_b2_w4, m6_b3_conv1_w, m6_b3_conv1_s, m6_b3_conv1_b, m6_b3_w2, m7_b0_conv1_w, m7_b0_conv1_s, m7_b0_conv1_b, m7_b0_w2, m7_b1_conv1_w, m7_b1_conv1_s, m7_b1_conv1_b, m7_b1_conv2_w, m7_b1_conv2_s, m7_b1_conv2_b, m7_b1_w3, m7_b1_b3, m7_b1_w4, m7_b2_conv1_w, m7_b2_conv1_s, m7_b2_conv1_b, m7_b2_conv2_w, m7_b2_conv2_s, m7_b2_conv2_b, m7_b2_w3, m7_b2_b3, m7_b2_w4, m7_b3_conv1_w, m7_b3_conv1_s, m7_b3_conv1_b, m7_b3_w2, m8_b0_conv1_w, m8_b0_conv1_s, m8_b0_conv1_b, m8_b0_w2, m8_b1_conv1_w, m8_b1_conv1_s, m8_b1_conv1_b, m8_b1_conv2_w, m8_b1_conv2_s, m8_b1_conv2_b, m8_b1_w3, m8_b1_b3, m8_b1_w4, m8_b2_conv1_w, m8_b2_conv1_s, m8_b2_conv1_b, m8_b2_conv2_w, m8_b2_conv2_s, m8_b2_conv2_b, m8_b2_w3, m8_b2_b3, m8_b2_w4, m8_b3_conv1_w, m8_b3_conv1_s, m8_b3_conv1_b, m8_b3_w2, x):
    ml = locals()
    mixed = []
    for i in range(9):
        p = f"m{i}_"
        mixed.append((
            (ml[p + "b0_conv1_w"], ml[p + "b0_conv1_s"],
             ml[p + "b0_conv1_b"], ml[p + "b0_w2"]),
            (ml[p + "b1_conv1_w"], ml[p + "b1_conv1_s"], ml[p + "b1_conv1_b"],
             ml[p + "b1_conv2_w"], ml[p + "b1_conv2_s"], ml[p + "b1_conv2_b"],
             ml[p + "b1_w3"], ml[p + "b1_b3"], ml[p + "b1_w4"]),
            (ml[p + "b2_conv1_w"], ml[p + "b2_conv1_s"], ml[p + "b2_conv1_b"],
             ml[p + "b2_conv2_w"], ml[p + "b2_conv2_s"], ml[p + "b2_conv2_b"],
             ml[p + "b2_w3"], ml[p + "b2_b3"], ml[p + "b2_w4"]),
            (ml[p + "b3_conv1_w"], ml[p + "b3_conv1_s"],
             ml[p + "b3_conv1_b"], ml[p + "b3_w2"]),
        ))

    N = x.shape[0]
    y = _conv_stem(x, stem_w, stem_s, stem_b)

    taps, dims = _pool_taps(y, (1, 3, 3), (1, 2, 2))
    pre_params = (pre_conv1_w, pre_conv1_s, pre_conv1_b, pre_conv2_w,
                  pre_conv2_s, pre_conv2_b, pre_w3, pre_b3, pre_w4)
    y, _ = _run_blocks(taps, taps.shape[0], dims,
                       [("pre", pre_params, (1, 3, 3))])
    y, dims = _subsample(y, dims, (1, 2, 2))

    y, _ = _run_blocks(y, 0, dims, [("mixed", mixed[0], None),
                                    ("mixed", mixed[1], (3, 3, 3))])
    y, dims = _subsample(y, dims, (2, 2, 2))

    y, _ = _run_blocks(y, 0, dims,
                       [("mixed", mixed[i], None) for i in range(2, 6)]
                       + [("mixed", mixed[6], (2, 2, 2))])
    y, dims = _subsample(y, dims, (2, 2, 2))

    y, c = _run_blocks(y, 0, dims, [("mixed", mixed[7], None),
                                    ("mixed", mixed[8], None)])
    T, H, W, rows, rows_p = dims
    return y[:, :rows, :c].reshape(N, T, H, W, c)


# speedup vs baseline: 1.0636x; 1.0488x over previous
import functools
import math

import jax
import jax.numpy as jnp
import numpy as np
from jax.experimental import pallas as pl
from jax.experimental.pallas import tpu as pltpu


def _ru(x, m):
    return (x + m - 1) // m * m


def _same_pads(size, k, stride):
    pad = max((math.ceil(size / stride) - 1) * stride + k - size, 0)
    if size % stride != 0:
        pad += 1
    return pad // 2, pad - pad // 2


SP9 = tuple((0, dh, dw) for dh in (-1, 0, 1) for dw in (-1, 0, 1))
TP3 = tuple((dt, 0, 0) for dt in (-1, 0, 1))
PL27 = tuple((dt, dh, dw) for dt in (-1, 0, 1)
             for dh in (-1, 0, 1) for dw in (-1, 0, 1))


def _stem_kernel(x_ref, w_ref, s_ref, b_ref, o_ref, acc_ref):
    k = pl.program_id(1)

    @pl.when(k == 0)
    def _():
        acc_ref[...] = jnp.zeros_like(acc_ref)

    acc_ref[...] += jnp.dot(x_ref[0, 0], w_ref[0],
                            preferred_element_type=jnp.float32)

    @pl.when(k == pl.num_programs(1) - 1)
    def _():
        o_ref[0] = jnp.maximum(acc_ref[...] * s_ref[...] + b_ref[...], 0.0)


@functools.lru_cache(maxsize=None)
def _build_stem(N, n_taps, R, L, Cl):
    return pl.pallas_call(
        _stem_kernel,
        out_shape=jax.ShapeDtypeStruct((N, R, Cl), jnp.float32),
        grid=(N, n_taps),
        in_specs=[
            pl.BlockSpec((1, 1, R, L), lambda n, k: (k, n, 0, 0)),
            pl.BlockSpec((1, L, Cl), lambda n, k: (k, 0, 0)),
            pl.BlockSpec((1, Cl), lambda n, k: (0, 0)),
            pl.BlockSpec((1, Cl), lambda n, k: (0, 0)),
        ],
        out_specs=pl.BlockSpec((1, R, Cl), lambda n, k: (n, 0, 0)),
        scratch_shapes=[pltpu.VMEM((R, Cl), jnp.float32)],
        compiler_params=pltpu.CompilerParams(
            dimension_semantics=("parallel", "arbitrary")),
    )


def _block_body(blocks, n_taps, T, H, W, rows_n, rows_p, *refs):
    HW = H * W
    o_ref = refs[-1]
    refs = refs[:-1]

    if n_taps:
        x = jnp.max(refs[0][:, 0], axis=0)
    else:
        x = refs[0][0]

    def rowid(width):
        return jax.lax.broadcasted_iota(jnp.int32, (rows_p, width), 0)

    def tapmask(dt, dh, dw, width):
        m = rowid(width)
        ok = m < rows_n
        if dt:
            t = m // HW
            ok &= jnp.logical_and(t + dt >= 0, t + dt < T)
        if dh:
            h = (m // W) % H
            ok &= jnp.logical_and(h + dh >= 0, h + dh < H)
        if dw:
            w = m % W
            ok &= jnp.logical_and(w + dw >= 0, w + dw < W)
        return ok

    def shift(a, r):
        if r == 0:
            return a
        z = jnp.zeros((abs(r), a.shape[1]), a.dtype)
        if r > 0:
            return jnp.concatenate([a[r:], z], axis=0)
        return jnp.concatenate([z, a[:r]], axis=0)

    def tap_val(a, dt, dh, dw):
        r = dt * HW + dh * W + dw
        return jnp.where(tapmask(dt, dh, dw, a.shape[1]), shift(a, r), 0.0)

    def conv_taps(a, taps, wv):
        parts = [tap_val(a, *tap) for tap in taps]
        xc = jnp.concatenate(parts, axis=1)
        return jnp.dot(xc, wv, preferred_element_type=jnp.float32)

    def cbact(y, sb_ref):
        sb = sb_ref[...]
        return jnp.maximum(y * sb[0:1] + sb[1:2], 0.0)

    def gate(a, wv):
        valid = rowid(a.shape[1]) < rows_n
        mean = jnp.sum(jnp.where(valid, a, 0.0), axis=0,
                       keepdims=True) * (1.0 / rows_n)
        g = jax.nn.sigmoid(
            jnp.dot(mean, wv, preferred_element_type=jnp.float32))
        return a * g

    def b1b2(a, r):
        c1w, c1sb, c2w, c2sb, w3, b3, w4 = r
        w1 = c1w[0, 0, 0]
        c1 = w1.shape[1]
        c2 = c2w.shape[-1]
        t = cbact(jnp.dot(a[:, :w1.shape[0]], w1,
                          preferred_element_type=jnp.float32), c1sb)
        w2 = c2w[0].reshape(9 * c1, c2)
        t = cbact(conv_taps(t, SP9, w2), c2sb)
        w3v = w3[:, 0, 0].reshape(3 * c2, c2)
        u = jnp.maximum(conv_taps(t, TP3, w3v) + b3[...], 0.0)
        return gate(u, w4[0, 0, 0])

    def b0b3(a, r, pool):
        c1w, c1sb, gw = r
        if pool:
            acc = None
            for tap in PL27:
                v = tap_val(a, *tap)
                acc = v if acc is None else jnp.maximum(acc, v)
            a = acc
        w1 = c1w[0, 0, 0]
        t = cbact(jnp.dot(a[:, :w1.shape[0]], w1,
                          preferred_element_type=jnp.float32), c1sb)
        return gate(t, gw[0, 0, 0])

    i = 1
    for kind, pool_k in blocks:
        if kind == "pre":
            y = b1b2(x, refs[i:i + 7])
            i += 7
        else:
            y0 = b0b3(x, refs[i:i + 3], False)
            y1 = b1b2(x, refs[i + 3:i + 10])
            y2 = b1b2(x, refs[i + 10:i + 17])
            y3 = b0b3(x, refs[i + 17:i + 20], True)
            i += 20
            y = jnp.concatenate([y0, y1, y2, y3], axis=1)
        if pool_k:
            acc = None
            for dt in range(pool_k[0]):
                for dh in range(pool_k[1]):
                    for dw in range(pool_k[2]):
                        v = tap_val(y, dt, dh, dw)
                        acc = v if acc is None else jnp.maximum(acc, v)
            y = acc
        x = y
    o_ref[0] = x


@functools.lru_cache(maxsize=None)
def _build_block(blocks, n_taps, N, T, H, W, rows_n, rows_p, Cin_p, Cout_p,
                 wshapes):
    body = functools.partial(_block_body, blocks, n_taps, T, H, W,
                             rows_n, rows_p)
    if n_taps:
        x_spec = pl.BlockSpec((n_taps, 1, rows_p, Cin_p),
                              lambda n: (0, n, 0, 0))
    else:
        x_spec = pl.BlockSpec((1, rows_p, Cin_p), lambda n: (n, 0, 0))
    def _zero_map(nd):
        return lambda n: (0,) * nd

    in_specs = [x_spec] + [pl.BlockSpec(s, _zero_map(len(s)))
                           for s in wshapes]
    return pl.pallas_call(
        body,
        out_shape=jax.ShapeDtypeStruct((N, rows_p, Cout_p), jnp.float32),
        grid=(N,),
        in_specs=in_specs,
        out_specs=pl.BlockSpec((1, rows_p, Cout_p), lambda n: (n, 0, 0)),
        compiler_params=pltpu.CompilerParams(
            dimension_semantics=("parallel",)),
    )


def _p2(a, r, c):
    return jnp.pad(a, ((0, r - a.shape[0]), (0, c - a.shape[1])))


def _pv(v, c):
    return jnp.pad(v.astype(jnp.float32), (0, c - v.shape[0])).reshape(1, c)


def _psb(s, b):
    return jnp.stack([s, b])


def _prep_b1b2(c1w, c1s, c1b, c2w, c2s, c2b, w3, b3, w4):
    arrs = [c1w, _psb(c1s, c1b), c2w, _psb(c2s, c2b),
            w3, b3.reshape(1, -1), w4]
    return arrs, c2w.shape[-1]


def _prep_b0b3(c1w, c1s, c1b, gw):
    return [c1w, _psb(c1s, c1b), gw], c1w.shape[-1]


def _pool_taps(y, ksize, stride):
    N, T, H, W, C = y.shape
    pads = [(0, 0)]
    for i, (dim, k, s) in enumerate(zip((T, H, W), ksize, stride)):
        pads.append(_same_pads(dim, k, s))
    pads.append((0, 0))
    yp = jnp.pad(y, pads)
    Tp, Hp, Wp = yp.shape[1:4]
    To = (Tp - ksize[0]) // stride[0] + 1
    Ho = (Hp - ksize[1]) // stride[1] + 1
    Wo = (Wp - ksize[2]) // stride[2] + 1
    taps = []
    for dt in range(ksize[0]):
        for dh in range(ksize[1]):
            for dw in range(ksize[2]):
                taps.append(yp[:, dt:dt + stride[0] * To:stride[0],
                               dh:dh + stride[1] * Ho:stride[1],
                               dw:dw + stride[2] * Wo:stride[2], :])
    rows = To * Ho * Wo
    rows_p = _ru(rows, 8)
    st = jnp.stack(taps, axis=0).reshape(len(taps), N, rows, C)
    st = jnp.pad(st, ((0, 0), (0, 0), (0, rows_p - rows), (0, 0)))
    return st, (To, Ho, Wo, rows, rows_p)


def _run_blocks(x, n_taps, dims, specs):
    T, H, W, rows_n, rows_p = dims
    if n_taps:
        N = x.shape[1]
        Cin_p = x.shape[3]
    else:
        N = x.shape[0]
        Cin_p = x.shape[2]
    arrs = []
    blocks = []
    c_true = None
    for kind, params, pool_k in specs:
        if kind == "pre":
            a, c_true = _prep_b1b2(*params)
        else:
            b0a, b0c = _prep_b0b3(*params[0])
            b1a, b1c = _prep_b1b2(*params[1])
            b2a, b2c = _prep_b1b2(*params[2])
            b3a, b3c = _prep_b0b3(*params[3])
            a = b0a + b1a + b2a + b3a
            c_true = b0c + b1c + b2c + b3c
        arrs.extend(a)
        blocks.append((kind, pool_k if pool_k else ()))
    wshapes = tuple(a.shape for a in arrs)
    fn = _build_block(tuple(blocks), n_taps, N, T, H, W, rows_n, rows_p,
                      Cin_p, c_true, wshapes)
    return fn(x, *arrs), c_true


@functools.lru_cache(maxsize=None)
def _stem_onehot(Wo, kW, Cin, Lp):
    oh = np.zeros((Lp, Wo * kW * Cin), np.float32)
    for wo in range(Wo):
        for dw in range(kW):
            for c in range(Cin):
                l = (2 * wo + dw) * Cin + c
                oh[l, (wo * kW + dw) * Cin + c] = 1.0
    return jnp.asarray(oh)


def _subsample(y, dims, stride):
    T, H, W, rows, rows_p = dims
    N, _, C = y.shape
    y5 = y[:, :rows].reshape(N, T, H, W, C)
    y5 = y5[:, ::stride[0], ::stride[1], ::stride[2]]
    To, Ho, Wo = y5.shape[1:4]
    r2 = To * Ho * Wo
    r2p = _ru(r2, 8)
    y2 = y5.reshape(N, r2, C)
    if r2p != r2:
        y2 = jnp.pad(y2, ((0, 0), (0, r2p - r2), (0, 0)))
    return y2, (To, Ho, Wo, r2, r2p)


def _conv_stem(x, w, s, b):
    N, T, H, W, Cin = x.shape
    kT, kH, kW, _, Cout = w.shape
    pt = _same_pads(T, kT, 2)
    ph = _same_pads(H, kH, 2)
    pw = _same_pads(W, kW, 2)
    xw = x.reshape(N, T, H, W * Cin)
    xw = jnp.pad(xw, ((0, 0), pt, ph, (pw[0] * Cin, pw[1] * Cin)))
    Tp, Hp = xw.shape[1:3]
    L = xw.shape[3]
    To = (Tp - kT) // 2 + 1
    Ho = (Hp - kH) // 2 + 1
    Wp = L // Cin
    Wo = (Wp - kW) // 2 + 1
    Lp = _ru(L, 128)
    xw = jnp.pad(xw, ((0, 0), (0, 0), (0, 0), (0, Lp - L)))
    taps = []
    for dt in range(kT):
        for dh in range(kH):
            taps.append(xw[:, dt:dt + 2 * To:2, dh:dh + 2 * Ho:2, :])
    n_taps = len(taps)
    R = To * Ho
    xp = jnp.stack(taps, axis=0).reshape(n_taps, N, R, Lp)
    xp = xp.astype(jnp.bfloat16)
    oh = _stem_onehot(Wo, kW, Cin, Lp)
    wf = w.reshape(kT * kH, kW * Cin, Cout)
    wbig = jnp.einsum("lwk,tkc->tlwc", oh.reshape(Lp, Wo, kW * Cin), wf)
    wbig = wbig.reshape(n_taps, Lp, Wo * Cout).astype(jnp.bfloat16)
    sv = jnp.tile(s.astype(jnp.float32), Wo).reshape(1, Wo * Cout)
    bv = jnp.tile(b.astype(jnp.float32), Wo).reshape(1, Wo * Cout)
    y = _build_stem(N, n_taps, R, Lp, Wo * Cout)(xp, wbig, sv, bv)
    return y.reshape(N, To, Ho, Wo, Cout)


def kernel(stem_w, stem_s, stem_b, pre_conv1_w, pre_conv1_s, pre_conv1_b, pre_conv2_w, pre_conv2_s, pre_conv2_b, pre_w3, pre_b3, pre_w4, m0_b0_conv1_w, m0_b0_conv1_s, m0_b0_conv1_b, m0_b0_w2, m0_b1_conv1_w, m0_b1_conv1_s, m0_b1_conv1_b, m0_b1_conv2_w, m0_b1_conv2_s, m0_b1_conv2_b, m0_b1_w3, m0_b1_b3, m0_b1_w4, m0_b2_conv1_w, m0_b2_conv1_s, m0_b2_conv1_b, m0_b2_conv2_w, m0_b2_conv2_s, m0_b2_conv2_b, m0_b2_w3, m0_b2_b3, m0_b2_w4, m0_b3_conv1_w, m0_b3_conv1_s, m0_b3_conv1_b, m0_b3_w2, m1_b0_conv1_w, m1_b0_conv1_s, m1_b0_conv1_b, m1_b0_w2, m1_b1_conv1_w, m1_b1_conv1_s, m1_b1_conv1_b, m1_b1_conv2_w, m1_b1_conv2_s, m1_b1_conv2_b, m1_b1_w3, m1_b1_b3, m1_b1_w4, m1_b2_conv1_w, m1_b2_conv1_s, m1_b2_conv1_b, m1_b2_conv2_w, m1_b2_conv2_s, m1_b2_conv2_b, m1_b2_w3, m1_b2_b3, m1_b2_w4, m1_b3_conv1_w, m1_b3_conv1_s, m1_b3_conv1_b, m1_b3_w2, m2_b0_conv1_w, m2_b0_conv1_s, m2_b0_conv1_b, m2_b0_w2, m2_b1_conv1_w, m2_b1_conv1_s, m2_b1_conv1_b, m2_b1_conv2_w, m2_b1_conv2_s, m2_b1_conv2_b, m2_b1_w3, m2_b1_b3, m2_b1_w4, m2_b2_conv1_w, m2_b2_conv1_s, m2_b2_conv1_b, m2_b2_conv2_w, m2_b2_conv2_s, m2_b2_conv2_b, m2_b2_w3, m2_b2_b3, m2_b2_w4, m2_b3_conv1_w, m2_b3_conv1_s, m2_b3_conv1_b, m2_b3_w2, m3_b0_conv1_w, m3_b0_conv1_s, m3_b0_conv1_b, m3_b0_w2, m3_b1_conv1_w, m3_b1_conv1_s, m3_b1_conv1_b, m3_b1_conv2_w, m3_b1_conv2_s, m3_b1_conv2_b, m3_b1_w3, m3_b1_b3, m3_b1_w4, m3_b2_conv1_w, m3_b2_conv1_s, m3_b2_conv1_b, m3_b2_conv2_w, m3_b2_conv2_s, m3_b2_conv2_b, m3_b2_w3, m3_b2_b3, m3_b2_w4, m3_b3_conv1_w, m3_b3_conv1_s, m3_b3_conv1_b, m3_b3_w2, m4_b0_conv1_w, m4_b0_conv1_s, m4_b0_conv1_b, m4_b0_w2, m4_b1_conv1_w, m4_b1_conv1_s, m4_b1_conv1_b, m4_b1_conv2_w, m4_b1_conv2_s, m4_b1_conv2_b, m4_b1_w3, m4_b1_b3, m4_b1_w4, m4_b2_conv1_w, m4_b2_conv1_s, m4_b2_conv1_b, m4_b2_conv2_w, m4_b2_conv2_s, m4_b2_conv2_b, m4_b2_w3, m4_b2_b3, m4_b2_w4, m4_b3_conv1_w, m4_b3_conv1_s, m4_b3_conv1_b, m4_b3_w2, m5_b0_conv1_w, m5_b0_conv1_s, m5_b0_conv1_b, m5_b0_w2, m5_b1_conv1_w, m5_b1_conv1_s, m5_b1_conv1_b, m5_b1_conv2_w, m5_b1_conv2_s, m5_b1_conv2_b, m5_b1_w3, m5_b1_b3, m5_b1_w4, m5_b2_conv1_w, m5_b2_conv1_s, m5_b2_conv1_b, m5_b2_conv2_w, m5_b2_conv2_s, m5_b2_conv2_b, m5_b2_w3, m5_b2_b3, m5_b2_w4, m5_b3_conv1_w, m5_b3_conv1_s, m5_b3_conv1_b, m5_b3_w2, m6_b0_conv1_w, m6_b0_conv1_s, m6_b0_conv1_b, m6_b0_w2, m6_b1_conv1_w, m6_b1_conv1_s, m6_b1_conv1_b, m6_b1_conv2_w, m6_b1_conv2_s, m6_b1_conv2_b, m6_b1_w3, m6_b1_b3, m6_b1_w4, m6_b2_conv1_w, m6_b2_conv1_s, m6_b2_conv1_b, m6_b2_conv2_w, m6_b2_conv2_s, m6_b2_conv2_b, m6_b2_w3, m6_b2_b3, m6_b2_w4, m6_b3_conv1_w, m6_b3_conv1_s, m6_b3_conv1_b, m6_b3_w2, m7_b0_conv1_w, m7_b0_conv1_s, m7_b0_conv1_b, m7_b0_w2, m7_b1_conv1_w, m7_b1_conv1_s, m7_b1_conv1_b, m7_b1_conv2_w, m7_b1_conv2_s, m7_b1_conv2_b, m7_b1_w3, m7_b1_b3, m7_b1_w4, m7_b2_conv1_w, m7_b2_conv1_s, m7_b2_conv1_b, m7_b2_conv2_w, m7_b2_conv2_s, m7_b2_conv2_b, m7_b2_w3, m7_b2_b3, m7_b2_w4, m7_b3_conv1_w, m7_b3_conv1_s, m7_b3_conv1_b, m7_b3_w2, m8_b0_conv1_w, m8_b0_conv1_s, m8_b0_conv1_b, m8_b0_w2, m8_b1_conv1_w, m8_b1_conv1_s, m8_b1_conv1_b, m8_b1_conv2_w, m8_b1_conv2_s, m8_b1_conv2_b, m8_b1_w3, m8_b1_b3, m8_b1_w4, m8_b2_conv1_w, m8_b2_conv1_s, m8_b2_conv1_b, m8_b2_conv2_w, m8_b2_conv2_s, m8_b2_conv2_b, m8_b2_w3, m8_b2_b3, m8_b2_w4, m8_b3_conv1_w, m8_b3_conv1_s, m8_b3_conv1_b, m8_b3_w2, x):
    ml = locals()
    mixed = []
    for i in range(9):
        p = f"m{i}_"
        mixed.append((
            (ml[p + "b0_conv1_w"], ml[p + "b0_conv1_s"],
             ml[p + "b0_conv1_b"], ml[p + "b0_w2"]),
            (ml[p + "b1_conv1_w"], ml[p + "b1_conv1_s"], ml[p + "b1_conv1_b"],
             ml[p + "b1_conv2_w"], ml[p + "b1_conv2_s"], ml[p + "b1_conv2_b"],
             ml[p + "b1_w3"], ml[p + "b1_b3"], ml[p + "b1_w4"]),
            (ml[p + "b2_conv1_w"], ml[p + "b2_conv1_s"], ml[p + "b2_conv1_b"],
             ml[p + "b2_conv2_w"], ml[p + "b2_conv2_s"], ml[p + "b2_conv2_b"],
             ml[p + "b2_w3"], ml[p + "b2_b3"], ml[p + "b2_w4"]),
            (ml[p + "b3_conv1_w"], ml[p + "b3_conv1_s"],
             ml[p + "b3_conv1_b"], ml[p + "b3_w2"]),
        ))

    N = x.shape[0]
    y = _conv_stem(x, stem_w, stem_s, stem_b)

    taps, dims = _pool_taps(y, (1, 3, 3), (1, 2, 2))
    pre_params = (pre_conv1_w, pre_conv1_s, pre_conv1_b, pre_conv2_w,
                  pre_conv2_s, pre_conv2_b, pre_w3, pre_b3, pre_w4)
    y, _ = _run_blocks(taps, taps.shape[0], dims,
                       [("pre", pre_params, (1, 3, 3))])
    y, dims = _subsample(y, dims, (1, 2, 2))

    y, _ = _run_blocks(y, 0, dims, [("mixed", mixed[0], None),
                                    ("mixed", mixed[1], (3, 3, 3))])
    y, dims = _subsample(y, dims, (2, 2, 2))

    y, _ = _run_blocks(y, 0, dims,
                       [("mixed", mixed[i], None) for i in range(2, 6)]
                       + [("mixed", mixed[6], (2, 2, 2))])
    y, dims = _subsample(y, dims, (2, 2, 2))

    y, c = _run_blocks(y, 0, dims, [("mixed", mixed[7], None),
                                    ("mixed", mixed[8], None)])
    T, H, W, rows, rows_p = dims
    return y[:, :rows, :c].reshape(N, T, H, W, c)
